# baseline (speedup 1.0000x reference)
import dataclasses
import functools

import jax
import jax.numpy as jnp
from jax import lax
from jax.experimental import pallas as pl
from jax.experimental.pallas import tpu as pltpu
from jax.experimental.pallas import tpu_sc as plsc

NUM_CORES = 2
NUM_SUBCORES = 16
NUM_WORKERS = NUM_CORES * NUM_SUBCORES
SPAN = 3200
CHUNK = 80
SNBUF = 3
SDEPTH = 2
TNBUF = 2
WIN = 8
LANES = 16


def kernel(input, node_segment):
    n = node_segment.shape[0]
    v, d = input.shape
    cps = SPAN // CHUNK
    full_workers = n // SPAN
    rem = n - full_workers * SPAN
    rem_chunks = rem // CHUNK
    assert SPAN % CHUNK == 0 and rem % CHUNK == 0
    assert SPAN % 8 == 0 and CHUNK % 8 == 0 and CHUNK >= LANES
    assert d % LANES == 0
    assert full_workers + (1 if rem else 0) == NUM_WORKERS
    min_chunks = rem_chunks if rem else cps
    assert (min_chunks + 1) // 2 >= SNBUF and min_chunks // 2 >= TNBUF

    idx = node_segment.astype(jnp.int32)
    mesh = plsc.VectorSubcoreMesh(core_axis_name="c", subcore_axis_name="s")

    cp = pltpu.CompilerParams()
    if "needs_layout_passes" in pltpu.CompilerParams.__dataclass_fields__:
        cp = dataclasses.replace(cp, needs_layout_passes=False)

    @functools.partial(
        pl.kernel,
        compiler_params=cp,
        out_type=jax.ShapeDtypeStruct((n, d), input.dtype),
        mesh=mesh,
        scratch_types=[
            pltpu.VMEM((SPAN,), jnp.int32),
            pltpu.VMEM((SNBUF, CHUNK, d), jnp.float32),
            pltpu.VMEM((TNBUF, CHUNK, d), jnp.float32),
            pltpu.VMEM((WIN, d), jnp.float32),
            pltpu.VMEM_SHARED((v, d), jnp.float32),
            pltpu.SemaphoreType.DMA,
            pltpu.SemaphoreType.DMA((SNBUF,)),
            pltpu.SemaphoreType.DMA((SNBUF,)),
            pltpu.SemaphoreType.DMA((TNBUF,)),
            pltpu.SemaphoreType.DMA,
        ],
    )
    def gather_kernel(table_hbm, idx_hbm, out_hbm, idx_all, srows_v, trows_v,
                      win_v, table_sh, sem_i, sem_g, sem_ws, sem_wt, sem_fb):
        sid = lax.axis_index("s")
        wid = sid * NUM_CORES + lax.axis_index("c")
        base = wid * SPAN

        @pl.when(wid < full_workers)
        def _():
            pltpu.async_copy(idx_hbm.at[pl.ds(base, SPAN)],
                             idx_all.at[pl.ds(0, SPAN)], sem_i)

        if rem:
            @pl.when(wid == full_workers)
            def _():
                pltpu.async_copy(idx_hbm.at[pl.ds(base, rem)],
                                 idx_all.at[pl.ds(0, rem)], sem_i)

        rows_per_sub = v // NUM_SUBCORES
        assert rows_per_sub * NUM_SUBCORES == v
        pltpu.sync_copy(table_hbm.at[pl.ds(sid * rows_per_sub, rows_per_sub)],
                        table_sh.at[pl.ds(sid * rows_per_sub, rows_per_sub)])
        plsc.subcore_barrier()

        @pl.when(wid < full_workers)
        def _():
            pltpu.make_async_copy(idx_hbm.at[pl.ds(base, SPAN)],
                                  idx_all.at[pl.ds(0, SPAN)], sem_i).wait()

        if rem:
            @pl.when(wid == full_workers)
            def _():
                pltpu.make_async_copy(idx_hbm.at[pl.ds(base, rem)],
                                      idx_all.at[pl.ds(0, rem)], sem_i).wait()

        def guarded(k, fn):
            if k < min_chunks:
                fn()
            else:
                pl.when(wid < full_workers)(fn)

        def idx_slice(k):
            return idx_all.at[pl.ds(k * CHUNK, CHUNK)]

        def start_gather(k):
            b = (k // 2) % SNBUF
            pltpu.async_copy(table_sh.at[idx_slice(k)], srows_v.at[b],
                             sem_g.at[b])

        def wait_swrite(b):
            pltpu.make_async_copy(
                srows_v.at[b], out_hbm.at[pl.ds(0, CHUNK)], sem_ws.at[b]).wait()

        def finish_stream_chunk(k):
            b = (k // 2) % SNBUF
            pltpu.make_async_copy(table_sh.at[idx_slice(k)], srows_v.at[b],
                                  sem_g.at[b]).wait()
            pltpu.async_copy(srows_v.at[b],
                             out_hbm.at[pl.ds(base + k * CHUNK, CHUNK)],
                             sem_ws.at[b])

        def wait_twrite(b):
            pltpu.make_async_copy(
                trows_v.at[b], out_hbm.at[pl.ds(0, CHUNK)], sem_wt.at[b]).wait()

        def tec_chunk(k):
            t = k // 2
            b = t % TNBUF
            if t >= TNBUF:
                wait_twrite(b)

            k0 = k * CHUNK
            head = idx_all[pl.ds(k0, LANES)]
            tail = idx_all[pl.ds(k0 + CHUNK - LANES, LANES)]
            lo = lax.reduce_min(head, (0,))
            hi = lax.reduce_max(tail, (0,))
            lo_c = jnp.minimum(lo, jnp.int32(v - WIN))
            narrow = (hi - lo_c) < WIN

            @pl.when(narrow)
            def _():
                pltpu.sync_copy(table_sh.at[pl.ds(lo_c, WIN)], win_v)
                offs_base = jnp.broadcast_to(lo_c, (LANES,))

                @pl.loop(0, CHUNK)
                def _(i):
                    row_sel = jnp.broadcast_to(jnp.int32(k0) + i, (LANES,))
                    offs = plsc.load_gather(idx_all, [row_sel]) - offs_base
                    for j in range(d // LANES):
                        col = lax.iota(jnp.int32, LANES) + jnp.int32(j * LANES)
                        trows_v[b, i, pl.ds(j * LANES, LANES)] = (
                            plsc.load_gather(win_v, [offs, col]))

            @pl.when(jnp.logical_not(narrow))
            def _():
                pltpu.async_copy(table_sh.at[idx_slice(k)], trows_v.at[b],
                                 sem_fb).wait()

            pltpu.async_copy(trows_v.at[b],
                             out_hbm.at[pl.ds(base + k * CHUNK, CHUNK)],
                             sem_wt.at[b])

        for e in range(min(SDEPTH, (cps + 1) // 2)):
            guarded(2 * e, lambda e=e: start_gather(2 * e))
        for k in range(cps):
            if k % 2 == 0:
                guarded(k, lambda k=k: finish_stream_chunk(k))
                j = k + 2 * SDEPTH
                if j < cps:
                    def advance(k=k, j=j):
                        e_next = j // 2
                        if e_next >= SNBUF:
                            wait_swrite((e_next - SNBUF) % SNBUF)
                        start_gather(j)
                    guarded(j, advance)
            else:
                guarded(k, lambda k=k: tec_chunk(k))

        for b in range(SNBUF):
            wait_swrite(b)
        for b in range(TNBUF):
            wait_twrite(b)

    return gather_kernel(input, idx)


# speedup vs baseline: 2.3135x; 2.3135x over previous
import functools

import jax
import jax.numpy as jnp
from jax import lax
from jax.experimental import pallas as pl
from jax.experimental.pallas import tpu as pltpu
from jax.experimental.pallas import tpu_sc as plsc

NUM_CORES = 2
NUM_SUBCORES = 16
NUM_WORKERS = NUM_CORES * NUM_SUBCORES
SPAN = 3200
CHUNK = 80
NBUF = 4
DEPTH = 3


def kernel(input, node_segment):
    n = node_segment.shape[0]
    v, d = input.shape
    cps = SPAN // CHUNK
    full_workers = n // SPAN
    rem = n - full_workers * SPAN
    rem_chunks = rem // CHUNK
    assert SPAN % CHUNK == 0 and rem % CHUNK == 0
    assert SPAN % 8 == 0 and CHUNK % 8 == 0
    assert full_workers + (1 if rem else 0) == NUM_WORKERS
    assert min(cps, rem_chunks if rem else cps) >= NBUF

    idx = node_segment.astype(jnp.int32)
    mesh = plsc.VectorSubcoreMesh(core_axis_name="c", subcore_axis_name="s")

    @functools.partial(
        pl.kernel,
        out_type=jax.ShapeDtypeStruct((n, d), input.dtype),
        mesh=mesh,
        scratch_types=[
            pltpu.VMEM((SPAN,), jnp.int32),
            pltpu.VMEM((NBUF, CHUNK, d), jnp.float32),
            pltpu.VMEM_SHARED((v, d), jnp.float32),
            pltpu.SemaphoreType.DMA,
            pltpu.SemaphoreType.DMA((NBUF,)),
            pltpu.SemaphoreType.DMA((NBUF,)),
        ],
    )
    def gather_kernel(table_hbm, idx_hbm, out_hbm, idx_all, rows_v, table_sh,
                      sem_i, sem_g, sem_w):
        sid = lax.axis_index("s")
        wid = sid * NUM_CORES + lax.axis_index("c")
        base = wid * SPAN

        @pl.when(wid < full_workers)
        def _():
            pltpu.async_copy(idx_hbm.at[pl.ds(base, SPAN)],
                             idx_all.at[pl.ds(0, SPAN)], sem_i)

        if rem:
            @pl.when(wid == full_workers)
            def _():
                pltpu.async_copy(idx_hbm.at[pl.ds(base, rem)],
                                 idx_all.at[pl.ds(0, rem)], sem_i)

        rows_per_sub = v // NUM_SUBCORES
        assert rows_per_sub * NUM_SUBCORES == v
        pltpu.sync_copy(table_hbm.at[pl.ds(sid * rows_per_sub, rows_per_sub)],
                        table_sh.at[pl.ds(sid * rows_per_sub, rows_per_sub)])
        plsc.subcore_barrier()

        @pl.when(wid < full_workers)
        def _():
            pltpu.make_async_copy(idx_hbm.at[pl.ds(base, SPAN)],
                                  idx_all.at[pl.ds(0, SPAN)], sem_i).wait()

        if rem:
            @pl.when(wid == full_workers)
            def _():
                pltpu.make_async_copy(idx_hbm.at[pl.ds(base, rem)],
                                      idx_all.at[pl.ds(0, rem)], sem_i).wait()

        def guarded(k, fn):
            if k < (rem_chunks if rem else cps):
                fn()
            else:
                pl.when(wid < full_workers)(fn)

        def start_gather(k):
            b = k % NBUF
            pltpu.async_copy(
                table_sh.at[idx_all.at[pl.ds(k * CHUNK, CHUNK)]],
                rows_v.at[b], sem_g.at[b])

        def wait_write(k):
            b = k % NBUF
            pltpu.make_async_copy(
                rows_v.at[b], out_hbm.at[pl.ds(0, CHUNK)], sem_w.at[b]).wait()

        def finish_chunk(k):
            b = k % NBUF
            pltpu.make_async_copy(
                table_sh.at[idx_all.at[pl.ds(k * CHUNK, CHUNK)]],
                rows_v.at[b], sem_g.at[b]).wait()
            pltpu.async_copy(
                rows_v.at[b], out_hbm.at[pl.ds(base + k * CHUNK, CHUNK)],
                sem_w.at[b])

        for k in range(min(DEPTH, cps)):
            guarded(k, lambda k=k: start_gather(k))
        for k in range(cps):
            guarded(k, lambda k=k: finish_chunk(k))
            j = k + DEPTH
            if j < cps:
                def advance(j=j):
                    if j >= NBUF:
                        wait_write(j - NBUF)
                    start_gather(j)
                guarded(j, advance)

        for b in range(NBUF):
            wait_write(b)

    return gather_kernel(input, idx)


# speedup vs baseline: 2.3372x; 1.0102x over previous
import functools

import jax
import jax.numpy as jnp
from jax import lax
from jax.experimental import pallas as pl
from jax.experimental.pallas import tpu as pltpu
from jax.experimental.pallas import tpu_sc as plsc

NUM_CORES = 2
NUM_SUBCORES = 16
NUM_WORKERS = NUM_CORES * NUM_SUBCORES
SPAN = 3200
CHUNK = 80
NBUF = 6
DEPTH = 5


def kernel(input, node_segment):
    n = node_segment.shape[0]
    v, d = input.shape
    cps = SPAN // CHUNK
    full_workers = n // SPAN
    rem = n - full_workers * SPAN
    rem_chunks = rem // CHUNK
    assert SPAN % CHUNK == 0 and rem % CHUNK == 0
    assert SPAN % 8 == 0 and CHUNK % 8 == 0
    assert full_workers + (1 if rem else 0) == NUM_WORKERS
    assert min(cps, rem_chunks if rem else cps) >= NBUF

    idx = node_segment.astype(jnp.int32)
    mesh = plsc.VectorSubcoreMesh(core_axis_name="c", subcore_axis_name="s")

    @functools.partial(
        pl.kernel,
        out_type=jax.ShapeDtypeStruct((n, d), input.dtype),
        mesh=mesh,
        scratch_types=[
            pltpu.VMEM((SPAN,), jnp.int32),
            pltpu.VMEM((NBUF, CHUNK, d), jnp.float32),
            pltpu.VMEM_SHARED((v, d), jnp.float32),
            pltpu.SemaphoreType.DMA,
            pltpu.SemaphoreType.DMA((NBUF,)),
            pltpu.SemaphoreType.DMA((NBUF,)),
        ],
    )
    def gather_kernel(table_hbm, idx_hbm, out_hbm, idx_all, rows_v, table_sh,
                      sem_i, sem_g, sem_w):
        sid = lax.axis_index("s")
        wid = sid * NUM_CORES + lax.axis_index("c")
        base = wid * SPAN

        @pl.when(wid < full_workers)
        def _():
            pltpu.async_copy(idx_hbm.at[pl.ds(base, SPAN)],
                             idx_all.at[pl.ds(0, SPAN)], sem_i)

        if rem:
            @pl.when(wid == full_workers)
            def _():
                pltpu.async_copy(idx_hbm.at[pl.ds(base, rem)],
                                 idx_all.at[pl.ds(0, rem)], sem_i)

        rows_per_sub = v // NUM_SUBCORES
        assert rows_per_sub * NUM_SUBCORES == v
        pltpu.sync_copy(table_hbm.at[pl.ds(sid * rows_per_sub, rows_per_sub)],
                        table_sh.at[pl.ds(sid * rows_per_sub, rows_per_sub)])
        plsc.subcore_barrier()

        @pl.when(wid < full_workers)
        def _():
            pltpu.make_async_copy(idx_hbm.at[pl.ds(base, SPAN)],
                                  idx_all.at[pl.ds(0, SPAN)], sem_i).wait()

        if rem:
            @pl.when(wid == full_workers)
            def _():
                pltpu.make_async_copy(idx_hbm.at[pl.ds(base, rem)],
                                      idx_all.at[pl.ds(0, rem)], sem_i).wait()

        def guarded(k, fn):
            if k < (rem_chunks if rem else cps):
                fn()
            else:
                pl.when(wid < full_workers)(fn)

        def start_gather(k):
            b = k % NBUF
            pltpu.async_copy(
                table_sh.at[idx_all.at[pl.ds(k * CHUNK, CHUNK)]],
                rows_v.at[b], sem_g.at[b])

        def wait_write(k):
            b = k % NBUF
            pltpu.make_async_copy(
                rows_v.at[b], out_hbm.at[pl.ds(0, CHUNK)], sem_w.at[b]).wait()

        def finish_chunk(k):
            b = k % NBUF
            pltpu.make_async_copy(
                table_sh.at[idx_all.at[pl.ds(k * CHUNK, CHUNK)]],
                rows_v.at[b], sem_g.at[b]).wait()
            pltpu.async_copy(
                rows_v.at[b], out_hbm.at[pl.ds(base + k * CHUNK, CHUNK)],
                sem_w.at[b])

        for k in range(min(DEPTH, cps)):
            guarded(k, lambda k=k: start_gather(k))
        for k in range(cps):
            guarded(k, lambda k=k: finish_chunk(k))
            j = k + DEPTH
            if j < cps:
                def advance(j=j):
                    if j >= NBUF:
                        wait_write(j - NBUF)
                    start_gather(j)
                guarded(j, advance)

        for b in range(NBUF):
            wait_write(b)

    return gather_kernel(input, idx)
